# baseline (speedup 1.0000x reference)
import math

import jax
import jax.numpy as jnp
from jax.experimental import pallas as pl
from jax.experimental.pallas import tpu as pltpu

NCLS = 722
_D1 = math.exp(-0.5)
_D2 = math.exp(-1.0)
_D3 = math.exp(-2.0)

CHUNK = 2048


def _chunk_part(x, tcols):
    tt = jnp.swapaxes(tcols, 0, 1)
    t = jnp.concatenate(
        [tt[:, r:r + 1] for r in range(tcols.shape[0])], axis=0)

    rowmax = jnp.max(x, axis=-1, keepdims=True)
    sumexp = jnp.sum(jnp.exp(x - rowmax), axis=-1, keepdims=True)
    lse = jnp.log(sumexp) + rowmax

    col = jax.lax.broadcasted_iota(jnp.int32, x.shape, 1)
    dist = jnp.abs(col - t)
    w = jnp.where(dist == 0, 1.0,
        jnp.where(dist == 1, _D1,
        jnp.where(dist == 2, _D2,
        jnp.where(dist == 3, _D3, 0.0))))
    wdot = jnp.sum(w * x, axis=-1, keepdims=True)
    sw = jnp.sum(w, axis=-1, keepdims=True)
    return jnp.sum(sw * lse - wdot, axis=0, keepdims=True)


def _make_body(n_chunks, trows):
    def body(pred_hbm, tgt_hbm, out_ref,
             x0, x1, t0, t1, sem0, sem1, sem2, sem3):
        xbufs, tbufs = (x0, x1), (t0, t1)
        xsems, tsems = (sem0, sem1), (sem2, sem3)

        def start(i):
            b = i % 2
            cx = pltpu.make_async_copy(
                pred_hbm.at[pl.ds(i * CHUNK, CHUNK), :], xbufs[b], xsems[b])
            ct = pltpu.make_async_copy(
                tgt_hbm.at[pl.ds(i * trows, trows), :], tbufs[b], tsems[b])
            cx.start()
            ct.start()
            return cx, ct

        def wait(cs):
            cs[0].wait()
            cs[1].wait()

        acc = jnp.zeros((1, 1), jnp.float32)
        pending = start(0)
        for i in range(n_chunks):
            cur = pending
            if i + 1 < n_chunks:
                pending = start(i + 1)
            wait(cur)
            b = i % 2
            acc = acc + _chunk_part(xbufs[b][...], tbufs[b][...])
        out_ref[...] = acc

    return body


def kernel(pred, target):
    B, T, C = pred.shape
    n_tok = B * T
    pred2 = pred.reshape(n_tok, C)
    tgt128 = target.astype(jnp.int32).reshape(n_tok // 128, 128)

    n_chunks = n_tok // CHUNK
    trows = CHUNK // 128

    out = pl.pallas_call(
        _make_body(n_chunks, trows),
        in_specs=[
            pl.BlockSpec(memory_space=pltpu.MemorySpace.HBM),
            pl.BlockSpec(memory_space=pltpu.MemorySpace.HBM),
        ],
        out_specs=pl.BlockSpec(memory_space=pltpu.MemorySpace.VMEM),
        out_shape=jax.ShapeDtypeStruct((1, 1), jnp.float32),
        scratch_shapes=[
            pltpu.VMEM((CHUNK, C), jnp.float32),
            pltpu.VMEM((CHUNK, C), jnp.float32),
            pltpu.VMEM((trows, 128), jnp.int32),
            pltpu.VMEM((trows, 128), jnp.int32),
            pltpu.SemaphoreType.DMA,
            pltpu.SemaphoreType.DMA,
            pltpu.SemaphoreType.DMA,
            pltpu.SemaphoreType.DMA,
        ],
    )(pred2, tgt128)
    return out[0, 0] / n_tok


# speedup vs baseline: 1.0167x; 1.0167x over previous
import math

import jax
import jax.numpy as jnp
from jax.experimental import pallas as pl

NCLS = 722
_DECAY1 = math.exp(-0.5)
_DECAY2 = math.exp(-1.0)
_DECAY3 = math.exp(-2.0)

TOK_BLOCK = 1024


def _ce_body(pred_ref, tgt_ref, out_ref):
    i = pl.program_id(0)
    x = pred_ref[...]
    t = tgt_ref[...]
    rowmax = jnp.max(x, axis=-1, keepdims=True)
    sumexp = jnp.sum(jnp.exp(x - rowmax), axis=-1, keepdims=True)
    lse = jnp.log(sumexp) + rowmax

    col = jax.lax.broadcasted_iota(jnp.int32, x.shape, 1)
    dist = jnp.abs(col - t)
    w = jnp.where(dist == 0, 1.0,
        jnp.where(dist == 1, _DECAY1,
        jnp.where(dist == 2, _DECAY2,
        jnp.where(dist == 3, _DECAY3, 0.0))))
    wdot = jnp.sum(w * x, axis=-1, keepdims=True)
    sw = jnp.sum(w, axis=-1, keepdims=True)

    part = jnp.sum(sw * lse - wdot, axis=0, keepdims=True)

    @pl.when(i == 0)
    def _init():
        out_ref[...] = part

    @pl.when(i > 0)
    def _acc():
        out_ref[...] += part


def kernel(pred, target):
    B, T, C = pred.shape
    n_tok = B * T
    pred2 = pred.reshape(n_tok, C)
    tgt2 = target.astype(jnp.int32).reshape(n_tok, 1)
    grid = n_tok // TOK_BLOCK

    total = pl.pallas_call(
        _ce_body,
        grid=(grid,),
        in_specs=[
            pl.BlockSpec((TOK_BLOCK, C), lambda i: (i, 0)),
            pl.BlockSpec((TOK_BLOCK, 1), lambda i: (i, 0)),
        ],
        out_specs=pl.BlockSpec((1, 1), lambda i: (0, 0)),
        out_shape=jax.ShapeDtypeStruct((1, 1), jnp.float32),
    )(pred2, tgt2)
    return total[0, 0] / n_tok
